# baseline (speedup 1.0000x reference)
import functools

import jax
import jax.numpy as jnp
from jax import lax
from jax.experimental import pallas as pl
from jax.experimental.pallas import tpu as pltpu
from jax.experimental.pallas import tpu_sc as plsc

BATCH = 16384
LANES = 16
CHUNK = 512


@functools.cache
def _build(num_workers: int, b_per_w: int):
    nch = b_per_w // CHUNK
    mesh = plsc.VectorSubcoreMesh(core_axis_name="c", subcore_axis_name="s")
    num_cores = mesh.num_cores

    @functools.partial(
        pl.kernel,
        mesh=mesh,
        out_type=jax.ShapeDtypeStruct((BATCH,), jnp.float32),
        scratch_types=[
            pltpu.VMEM((b_per_w,), jnp.int32),
            pltpu.VMEM((b_per_w,), jnp.int32),
            pltpu.VMEM((b_per_w,), jnp.float32),
            pltpu.VMEM((b_per_w,), jnp.float32),
            pltpu.VMEM((LANES,), jnp.float32),
            pltpu.SemaphoreType.DMA,
            pltpu.SemaphoreType.DMA,
        ],
    )
    def k(user_hbm, mission_hbm, ut_hbm, mt_hbm, bias_hbm, out_hbm,
          uidx_v, midx_v, uval_v, mval_v, bias_v, sem_u, sem_m):
        wid = lax.axis_index("s") * num_cores + lax.axis_index("c")
        base = wid * b_per_w
        ld_u = pltpu.async_copy(user_hbm.at[pl.ds(base, b_per_w)], uidx_v, sem_u)
        ld_m = pltpu.async_copy(mission_hbm.at[pl.ds(base, b_per_w)], midx_v, sem_m)
        pltpu.sync_copy(bias_hbm, bias_v)
        copies = []
        ld_u.wait()
        for c in range(nch):
            s = pl.ds(c * CHUNK, CHUNK)
            copies.append(
                pltpu.async_copy(ut_hbm.at[uidx_v.at[s]], uval_v.at[s], sem_u))
        ld_m.wait()
        for c in range(nch):
            s = pl.ds(c * CHUNK, CHUNK)
            copies.append(
                pltpu.async_copy(mt_hbm.at[midx_v.at[s]], mval_v.at[s], sem_m))
        for cp in copies:
            cp.wait()
        bv = bias_v[...]
        for i in range(b_per_w // LANES):
            s = pl.ds(i * LANES, LANES)
            uval_v[s] = uval_v[s] + mval_v[s] + bv
        pltpu.sync_copy(uval_v, out_hbm.at[pl.ds(base, b_per_w)])

    return k


def kernel(user, mission, user_table, mission_table, bias):
    info = plsc.get_sparse_core_info()
    num_workers = info.num_cores * info.num_subcores
    b_per_w = BATCH // num_workers
    k = _build(num_workers, b_per_w)
    return k(
        user.astype(jnp.int32),
        mission.astype(jnp.int32),
        user_table.reshape(-1),
        mission_table.reshape(-1),
        jnp.broadcast_to(bias, (LANES,)),
    )


# speedup vs baseline: 1.0290x; 1.0290x over previous
import functools

import jax
import jax.numpy as jnp
from jax import lax
from jax.experimental import pallas as pl
from jax.experimental.pallas import tpu as pltpu
from jax.experimental.pallas import tpu_sc as plsc

BATCH = 16384
LANES = 16
CHUNK = 512


@functools.cache
def _build(num_workers: int, b_per_w: int):
    nch = b_per_w // CHUNK
    mesh = plsc.VectorSubcoreMesh(core_axis_name="c", subcore_axis_name="s")
    num_cores = mesh.num_cores

    @functools.partial(
        pl.kernel,
        mesh=mesh,
        out_type=jax.ShapeDtypeStruct((BATCH,), jnp.float32),
        scratch_types=[
            pltpu.VMEM((b_per_w,), jnp.int32),
            pltpu.VMEM((b_per_w,), jnp.int32),
            pltpu.VMEM((b_per_w,), jnp.float32),
            pltpu.VMEM((b_per_w,), jnp.float32),
            pltpu.VMEM((LANES,), jnp.float32),
            pltpu.SemaphoreType.DMA,
            pltpu.SemaphoreType.DMA,
        ],
    )
    def k(user_hbm, mission_hbm, ut_hbm, mt_hbm, bias_hbm, out_hbm,
          uidx_v, midx_v, uval_v, mval_v, bias_v, sem_u, sem_m):
        wid = lax.axis_index("s") * num_cores + lax.axis_index("c")
        base = wid * b_per_w
        ld_u = pltpu.async_copy(user_hbm.at[pl.ds(base, b_per_w)], uidx_v, sem_u)
        ld_m = pltpu.async_copy(mission_hbm.at[pl.ds(base, b_per_w)], midx_v, sem_m)
        pltpu.sync_copy(bias_hbm, bias_v.at[pl.ds(0, 1)])
        copies = []
        ld_u.wait()
        for c in range(nch):
            s = pl.ds(c * CHUNK, CHUNK)
            copies.append(
                pltpu.async_copy(ut_hbm.at[uidx_v.at[s]], uval_v.at[s], sem_u))
        ld_m.wait()
        for c in range(nch):
            s = pl.ds(c * CHUNK, CHUNK)
            copies.append(
                pltpu.async_copy(mt_hbm.at[midx_v.at[s]], mval_v.at[s], sem_m))
        for cp in copies:
            cp.wait()
        bv = bias_v[...][0]
        for i in range(b_per_w // LANES):
            s = pl.ds(i * LANES, LANES)
            uval_v[s] = uval_v[s] + mval_v[s] + bv
        pltpu.sync_copy(uval_v, out_hbm.at[pl.ds(base, b_per_w)])

    return k


def kernel(user, mission, user_table, mission_table, bias):
    info = plsc.get_sparse_core_info()
    num_workers = info.num_cores * info.num_subcores
    b_per_w = BATCH // num_workers
    k = _build(num_workers, b_per_w)
    return k(
        user,
        mission,
        user_table.reshape(-1),
        mission_table.reshape(-1),
        bias,
    )
